# baseline (speedup 1.0000x reference)
import functools

import jax
import jax.numpy as jnp
from jax import lax
from jax.experimental import pallas as pl
from jax.experimental.pallas import tpu as pltpu
from jax.experimental.pallas import tpu_sc as plsc

N_VALUES = 65536
TOTAL_Q = 2048 * 4096
NUM_CORES = 2
NUM_SUBCORES = 16
LANES = 16
NUM_WORKERS = NUM_CORES * NUM_SUBCORES
PER_WORKER = TOTAL_Q // NUM_WORKERS
CHUNK = 8192
NUM_CHUNKS = PER_WORKER // CHUNK
UNROLL = 8
N_BUCKETS = 16384
N_B = 16512


def _rot(x: int) -> int:
    return ((x << 4) | (x >> 12)) & 0xFFFF


def _search_pos_batch(pv_v, aux_v, lane_consts, qs):
    nu = len(qs)
    t4s = [jnp.zeros((LANES,), jnp.int32) for _ in range(nu)]
    for b in range(3, -1, -1):
        probes = [plsc.load_gather(aux_v, [t | lane_consts[b]]) for t in t4s]
        t4s = [
            jnp.where(probes[u] < qs[u], t4s[u] | (1 << (b + 4)), t4s[u])
            for u in range(nu)
        ]
    rposs = [t >> 4 for t in t4s]
    for bit in range(11, -1, -1):
        step = 1 << bit
        probes = [plsc.load_gather(pv_v, [r | _rot(step - 1)]) for r in rposs]
        rposs = [
            jnp.where(probes[u] < qs[u], rposs[u] | _rot(step), rposs[u])
            for u in range(nu)
        ]
    return [(r >> 4) | ((r & 0xF) << 12) for r in rposs]


@functools.partial(
    pl.kernel,
    mesh=plsc.VectorSubcoreMesh(core_axis_name="c", subcore_axis_name="s"),
    out_type=jax.ShapeDtypeStruct((TOTAL_Q,), jnp.int32),
    compiler_params=pltpu.CompilerParams(needs_layout_passes=False),
    scratch_types=[
        pltpu.VMEM((N_VALUES,), jnp.float32),
        pltpu.VMEM((N_B,), jnp.int32),
        pltpu.VMEM((16 * LANES,), jnp.float32),
        pltpu.VMEM((CHUNK,), jnp.float32),
        pltpu.VMEM((CHUNK,), jnp.float32),
        pltpu.VMEM((CHUNK,), jnp.int32),
        pltpu.VMEM((CHUNK,), jnp.int32),
        pltpu.SemaphoreType.DMA,
        pltpu.SemaphoreType.DMA,
        pltpu.SemaphoreType.DMA,
        pltpu.SemaphoreType.DMA,
    ],
)
def _sc_lookup(query_hbm, values_hbm, pv_hbm, out_hbm, pv_v, b_v, aux_v,
               q_v0, q_v1, o_v0, o_v1, si0, si1, so0, so1):
    wid = lax.axis_index("s") * NUM_CORES + lax.axis_index("c")
    base_w = wid * PER_WORKER

    pltpu.async_copy(
        query_hbm.at[pl.ds(base_w, CHUNK)], q_v0, si0
    )

    pltpu.sync_copy(pv_hbm, pv_v)
    for k in range(16):
        bk = plsc.load_gather(
            pv_v, [jnp.full((LANES,), 65520 + k, jnp.int32)]
        )
        aux_v[pl.ds(k * LANES, LANES)] = bk
    lane = lax.iota(jnp.int32, 16)
    lane_consts = [lane | (((1 << b) - 1) << 4) for b in range(4)]
    inv_scale = jnp.float32(1.0 / N_BUCKETS)

    def b_body(jb, carry):
        base = jb * (LANES * UNROLL)
        gs = [
            (lane + (base + u * LANES)).astype(jnp.float32) * inv_scale
            for u in range(UNROLL)
        ]
        poss = _search_pos_batch(pv_v, aux_v, lane_consts, gs)
        for u in range(UNROLL):
            b_v[pl.ds(base + u * LANES, LANES)] = poss[u]
        return carry

    lax.fori_loop(0, N_B // (LANES * UNROLL), b_body, 0)

    def w_body(j, wmax):
        a = b_v[pl.ds(j * LANES, LANES)]
        b = plsc.load_gather(b_v, [lane + (j * LANES + 1)])
        return jnp.maximum(wmax, b - a)

    wvec = lax.fori_loop(
        0, N_BUCKETS // LANES, w_body, jnp.zeros((LANES,), jnp.int32)
    )
    wc = jnp.max(wvec, axis=0) + 2
    m = jnp.int32(0)
    for t in range(16):
        m = m + jnp.where(jnp.int32(1 << t) < wc, 1, 0).astype(jnp.int32)
    mm = jnp.maximum(m, 5)
    extra = mm - 5
    step0 = (jnp.int32(1) << mm) >> 1
    lo_cap = jnp.int32(N_VALUES) - (jnp.int32(1) << mm)

    pltpu.sync_copy(values_hbm, pv_v)

    q_bufs = (q_v0, q_v1)
    o_bufs = (o_v0, o_v1)
    si = (si0, si1)
    so = (so0, so1)
    scale = jnp.float32(N_BUCKETS)

    def in_slice(g):
        return query_hbm.at[pl.ds(base_w + g * CHUNK, CHUNK)]

    def out_slice(g):
        return out_hbm.at[pl.ds(base_w + g * CHUNK, CHUNK)]

    def compute_chunk(q_v, o_v):
        def make_vec_body(dynamic_levels):
            def vec_body(j, inner_carry):
                off = j * (LANES * UNROLL)
                qs = [
                    q_v[pl.ds(off + u * LANES, LANES)] for u in range(UNROLL)
                ]
                ks = [
                    jnp.clip((q * scale).astype(jnp.int32), 0, N_BUCKETS - 1)
                    for q in qs
                ]
                los = [plsc.load_gather(b_v, [k]) for k in ks]
                poss = [jnp.minimum(l, lo_cap) for l in los]

                if dynamic_levels:
                    def lvl_body(t, carry):
                        step = carry[0]
                        ps = list(carry[1:])
                        stepm1 = step - 1
                        probes = [
                            plsc.load_gather(pv_v, [ps[u] + stepm1])
                            for u in range(UNROLL)
                        ]
                        ps = [
                            jnp.where(probes[u] < qs[u], ps[u] + step, ps[u])
                            for u in range(UNROLL)
                        ]
                        return (step >> 1, *ps)

                    res = lax.fori_loop(
                        0, extra, lvl_body, (jnp.full((LANES,), step0), *poss)
                    )
                    poss = list(res[1:])
                for step in (16, 8, 4, 2, 1):
                    probes = [
                        plsc.load_gather(pv_v, [p + (step - 1)]) for p in poss
                    ]
                    poss = [
                        jnp.where(probes[u] < qs[u], poss[u] + step, poss[u])
                        for u in range(UNROLL)
                    ]
                iss = [jnp.clip(p, 1, N_VALUES - 1) for p in poss]
                lefts = [plsc.load_gather(pv_v, [i - 1]) for i in iss]
                rights = [plsc.load_gather(pv_v, [i]) for i in iss]
                for u in range(UNROLL):
                    i = iss[u]
                    idx = jnp.where(
                        qs[u] - lefts[u] <= rights[u] - qs[u], i - 1, i
                    )
                    o_v[pl.ds(off + u * LANES, LANES)] = idx
                return inner_carry

            return vec_body

        n_vec = CHUNK // (LANES * UNROLL)

        @pl.when(extra == 0)
        def _():
            fast_body = make_vec_body(False)

            @plsc.parallel_loop(0, n_vec, unroll=2)
            def _(j):
                fast_body(j, 0)

        @pl.when(extra != 0)
        def _():
            lax.fori_loop(0, n_vec, make_vec_body(True), 0)

    def pair_body(p, carry):
        for b in range(2):
            g = p * 2 + b
            pltpu.make_async_copy(in_slice(g), q_bufs[b], si[b]).wait()
            if b == 0:
                pltpu.async_copy(in_slice(g + 1), q_bufs[1], si[1])
            else:
                @pl.when(p < NUM_CHUNKS // 2 - 1)
                def _():
                    pltpu.async_copy(in_slice(g + 1), q_bufs[0], si[0])

            @pl.when(p >= 1)
            def _():
                pltpu.make_async_copy(
                    o_bufs[b], out_slice(g - 2), so[b]
                ).wait()

            compute_chunk(q_bufs[b], o_bufs[b])
            pltpu.async_copy(o_bufs[b], out_slice(g), so[b])
        return carry

    lax.fori_loop(0, NUM_CHUNKS // 2, pair_body, 0)
    pltpu.make_async_copy(o_v0, out_slice(NUM_CHUNKS - 2), so0).wait()
    pltpu.make_async_copy(o_v1, out_slice(NUM_CHUNKS - 1), so1).wait()


@jax.jit
def kernel(query, values):
    pv = values.reshape(16, 4096).T.reshape(-1)
    out = _sc_lookup(query.reshape(-1), values, pv)
    return out.reshape(query.shape)


# speedup vs baseline: 1.1714x; 1.1714x over previous
import functools

import jax
import jax.numpy as jnp
from jax import lax
from jax.experimental import pallas as pl
from jax.experimental.pallas import tpu as pltpu
from jax.experimental.pallas import tpu_sc as plsc

N_VALUES = 65536
TOTAL_Q = 2048 * 4096
NUM_CORES = 2
NUM_SUBCORES = 16
LANES = 16
NUM_WORKERS = NUM_CORES * NUM_SUBCORES
PER_WORKER = TOTAL_Q // NUM_WORKERS
CHUNK = 8192
NUM_CHUNKS = PER_WORKER // CHUNK
UNROLL = 8
N_BUCKETS = 16384
N_B = 16512


def _rot(x: int) -> int:
    return ((x << 4) | (x >> 12)) & 0xFFFF


def _search_pos_batch(pv_v, aux_v, lane_consts, qs):
    nu = len(qs)
    t4s = [jnp.zeros((LANES,), jnp.int32) for _ in range(nu)]
    for b in range(3, -1, -1):
        probes = [plsc.load_gather(aux_v, [t | lane_consts[b]]) for t in t4s]
        t4s = [
            jnp.where(probes[u] < qs[u], t4s[u] | (1 << (b + 4)), t4s[u])
            for u in range(nu)
        ]
    rposs = [t >> 4 for t in t4s]
    for bit in range(11, -1, -1):
        step = 1 << bit
        probes = [plsc.load_gather(pv_v, [r | _rot(step - 1)]) for r in rposs]
        rposs = [
            jnp.where(probes[u] < qs[u], rposs[u] | _rot(step), rposs[u])
            for u in range(nu)
        ]
    return [(r >> 4) | ((r & 0xF) << 12) for r in rposs]


@functools.partial(
    pl.kernel,
    mesh=plsc.VectorSubcoreMesh(core_axis_name="c", subcore_axis_name="s"),
    out_type=jax.ShapeDtypeStruct((TOTAL_Q,), jnp.int32),
    compiler_params=pltpu.CompilerParams(needs_layout_passes=False),
    scratch_types=[
        pltpu.VMEM((N_VALUES,), jnp.float32),
        pltpu.VMEM((N_B,), jnp.int32),
        pltpu.VMEM((16 * LANES,), jnp.float32),
        pltpu.VMEM((CHUNK,), jnp.float32),
        pltpu.VMEM((CHUNK,), jnp.float32),
        pltpu.VMEM((CHUNK,), jnp.int32),
        pltpu.VMEM((CHUNK,), jnp.int32),
        pltpu.SemaphoreType.DMA,
        pltpu.SemaphoreType.DMA,
        pltpu.SemaphoreType.DMA,
        pltpu.SemaphoreType.DMA,
    ],
)
def _sc_lookup(query_hbm, values_hbm, pv_hbm, out_hbm, pv_v, b_v, aux_v,
               q_v0, q_v1, o_v0, o_v1, si0, si1, so0, so1):
    wid = lax.axis_index("s") * NUM_CORES + lax.axis_index("c")
    base_w = wid * PER_WORKER

    pltpu.async_copy(
        query_hbm.at[pl.ds(base_w, CHUNK)], q_v0, si0
    )

    pltpu.sync_copy(pv_hbm, pv_v)
    for k in range(16):
        bk = plsc.load_gather(
            pv_v, [jnp.full((LANES,), 65520 + k, jnp.int32)]
        )
        aux_v[pl.ds(k * LANES, LANES)] = bk
    lane = lax.iota(jnp.int32, 16)
    lane_consts = [lane | (((1 << b) - 1) << 4) for b in range(4)]
    inv_scale = jnp.float32(1.0 / N_BUCKETS)

    def b_body(jb, carry):
        base = jb * (LANES * UNROLL)
        gs = [
            (lane + (base + u * LANES)).astype(jnp.float32) * inv_scale
            for u in range(UNROLL)
        ]
        poss = _search_pos_batch(pv_v, aux_v, lane_consts, gs)
        for u in range(UNROLL):
            b_v[pl.ds(base + u * LANES, LANES)] = poss[u]
        return carry

    lax.fori_loop(0, N_B // (LANES * UNROLL), b_body, 0)

    def w_body(j, wmax):
        a = b_v[pl.ds(j * LANES, LANES)]
        b = plsc.load_gather(b_v, [lane + (j * LANES + 1)])
        return jnp.maximum(wmax, b - a)

    wvec = lax.fori_loop(
        0, N_BUCKETS // LANES, w_body, jnp.zeros((LANES,), jnp.int32)
    )
    wc = jnp.max(wvec, axis=0) + 2
    m = jnp.int32(0)
    for t in range(16):
        m = m + jnp.where(jnp.int32(1 << t) < wc, 1, 0).astype(jnp.int32)
    mm = jnp.maximum(m, 5)
    extra = mm - 5
    step0 = (jnp.int32(1) << mm) >> 1
    lo_cap = jnp.int32(N_VALUES) - (jnp.int32(1) << mm)

    pltpu.sync_copy(values_hbm, pv_v)

    q_bufs = (q_v0, q_v1)
    o_bufs = (o_v0, o_v1)
    si = (si0, si1)
    so = (so0, so1)
    scale = jnp.float32(N_BUCKETS)

    def in_slice(g):
        return query_hbm.at[pl.ds(base_w + g * CHUNK, CHUNK)]

    def out_slice(g):
        return out_hbm.at[pl.ds(base_w + g * CHUNK, CHUNK)]

    def compute_chunk(q_v, o_v):
        def make_vec_body(dynamic_levels):
            def vec_body(j, inner_carry):
                off = j * (LANES * UNROLL)
                qs = [
                    q_v[pl.ds(off + u * LANES, LANES)] for u in range(UNROLL)
                ]
                ks = [
                    jnp.clip((q * scale).astype(jnp.int32), 0, N_BUCKETS - 1)
                    for q in qs
                ]
                los = [plsc.load_gather(b_v, [k]) for k in ks]
                poss = [jnp.minimum(l, lo_cap) for l in los]

                if dynamic_levels:
                    def lvl_body(t, carry):
                        step = carry[0]
                        ps = list(carry[1:])
                        stepm1 = step - 1
                        probes = [
                            plsc.load_gather(pv_v, [ps[u] + stepm1])
                            for u in range(UNROLL)
                        ]
                        ps = [
                            jnp.where(probes[u] < qs[u], ps[u] + step, ps[u])
                            for u in range(UNROLL)
                        ]
                        return (step >> 1, *ps)

                    res = lax.fori_loop(
                        0, extra, lvl_body, (jnp.full((LANES,), step0), *poss)
                    )
                    poss = list(res[1:])
                for step in (16, 8, 4, 2, 1):
                    probes = [
                        plsc.load_gather(pv_v, [p + (step - 1)]) for p in poss
                    ]
                    poss = [
                        jnp.where(probes[u] < qs[u], poss[u] + step, poss[u])
                        for u in range(UNROLL)
                    ]
                iss = [jnp.clip(p, 1, N_VALUES - 1) for p in poss]
                lefts = [plsc.load_gather(pv_v, [i - 1]) for i in iss]
                rights = [plsc.load_gather(pv_v, [i]) for i in iss]
                for u in range(UNROLL):
                    i = iss[u]
                    idx = jnp.where(
                        qs[u] - lefts[u] <= rights[u] - qs[u], i - 1, i
                    )
                    o_v[pl.ds(off + u * LANES, LANES)] = idx
                return inner_carry

            return vec_body

        n_vec = CHUNK // (LANES * UNROLL)

        @pl.when(extra == 0)
        def _():
            fast_body = make_vec_body(False)

            @plsc.parallel_loop(0, n_vec, unroll=1)
            def _(j):
                fast_body(j, 0)

        @pl.when(extra != 0)
        def _():
            lax.fori_loop(0, n_vec, make_vec_body(True), 0)

    def pair_body(p, carry):
        for b in range(2):
            g = p * 2 + b
            pltpu.make_async_copy(in_slice(g), q_bufs[b], si[b]).wait()
            if b == 0:
                pltpu.async_copy(in_slice(g + 1), q_bufs[1], si[1])
            else:
                @pl.when(p < NUM_CHUNKS // 2 - 1)
                def _():
                    pltpu.async_copy(in_slice(g + 1), q_bufs[0], si[0])

            @pl.when(p >= 1)
            def _():
                pltpu.make_async_copy(
                    o_bufs[b], out_slice(g - 2), so[b]
                ).wait()

            compute_chunk(q_bufs[b], o_bufs[b])
            pltpu.async_copy(o_bufs[b], out_slice(g), so[b])
        return carry

    lax.fori_loop(0, NUM_CHUNKS // 2, pair_body, 0)
    pltpu.make_async_copy(o_v0, out_slice(NUM_CHUNKS - 2), so0).wait()
    pltpu.make_async_copy(o_v1, out_slice(NUM_CHUNKS - 1), so1).wait()


@jax.jit
def kernel(query, values):
    pv = values.reshape(16, 4096).T.reshape(-1)
    out = _sc_lookup(query.reshape(-1), values, pv)
    return out.reshape(query.shape)
